# Initial kernel scaffold; baseline (speedup 1.0000x reference)
import jax
import jax.numpy as jnp
from jax.experimental import pallas as pl


def kernel(x, edge_index, W, b, temp):
    raise NotImplementedError("write your pallas kernel here")


# speedup vs baseline: 2.5899x; 2.5899x over previous
import math

import jax
import jax.numpy as jnp
import numpy as np
from jax import lax
from jax.experimental import pallas as pl
from jax.experimental.pallas import tpu as pltpu
from jax.experimental.pallas import tpu_sc as plsc

K = 10
N = 10000
E = 160000
D = 256

NC = 2
NS = 16
L = 16
F = D // NC
FV = F // L

NP = 10240
ROWS_PS = NP // NS
CCH = 32
NCCH = ROWS_PS // CCH

ECH = 64
ROWS_E = 80
EP = ROWS_E * NS * 2 * ECH
PAD_IDX = N
SHIFT = 14
MASK = (1 << SHIFT) - 1


def _cheby(i, x):
    if i == 0:
        return 1.0
    if i == 1:
        return x
    t0, t1 = 1.0, x
    for _ in range(2, i + 1):
        t0, t1 = t1, 2.0 * x * t1 - t0
    return t1


def _cheb_t_padded():
    xs = [math.cos((K - j + 0.5) * math.pi / (K + 1)) for j in range(K + 1)]
    Ct = np.zeros((128, 128), dtype=np.float32)
    for i in range(K + 1):
        for j in range(K + 1):
            Ct[j, i] = _cheby(i, xs[j])
    return Ct


_CHEB_T = _cheb_t_padded()


def _mm_body(x_ref, w_ref, b_ref, o_ref):
    o_ref[...] = (
        lax.dot_general(
            x_ref[...], w_ref[...], (((1,), (1,)), ((), ())),
            preferred_element_type=jnp.float32,
        )
        + b_ref[...]
    )


def _matmul_h2(x_pad, W, b2):
    nb = NP // 128
    return pl.pallas_call(
        _mm_body,
        grid=(nb, NC),
        in_specs=[
            pl.BlockSpec((128, D), lambda i, c: (i, 0)),
            pl.BlockSpec((F, D), lambda i, c: (c, 0)),
            pl.BlockSpec((1, F), lambda i, c: (0, c)),
        ],
        out_specs=pl.BlockSpec((128, F), lambda i, c: (c * nb + i, 0)),
        out_shape=jax.ShapeDtypeStruct((NC * NP, F), jnp.float32),
    )(x_pad, W, b2)


def _coe_body(t_ref, c_ref, o_ref):
    tr = jnp.maximum(t_ref[...], 0.0)
    o_ref[...] = lax.dot_general(
        tr, c_ref[...], (((1,), (0,)), ((), ())),
        preferred_element_type=jnp.float32,
    ) * (2.0 / (K + 1))


def _coe_tc(temp8, chebT):
    return pl.pallas_call(
        _coe_body,
        out_shape=jax.ShapeDtypeStruct((8, 128), jnp.float32),
    )(temp8, chebT)


def _ep_body(a_ref, b_ref, o_ref):
    o_ref[:, :F] = jnp.maximum(a_ref[...], 0.0)
    o_ref[:, F:] = jnp.maximum(b_ref[...], 0.0)


def _epilogue(out2):
    nb = NP // 128
    return pl.pallas_call(
        _ep_body,
        grid=(nb,),
        in_specs=[
            pl.BlockSpec((128, F), lambda i: (i, 0)),
            pl.BlockSpec((128, F), lambda i: (nb + i, 0)),
        ],
        out_specs=pl.BlockSpec((128, D), lambda i: (i, 0)),
        out_shape=jax.ShapeDtypeStruct((NP, D), jnp.float32),
    )(out2, out2)


def _splat(ref, idx):
    return plsc.load_gather(ref, [jnp.full((L,), idx, jnp.int32)])


def _sc_body(h2, packed, coep,
             out2, txP, txQ, zA, zB,
             acc,
             pk_my, sidx, didx, gbuf, outv, zv,
             dis):
    c = lax.axis_index("c")
    s = lax.axis_index("s")
    core_off = c * NP
    zeros16 = jnp.zeros((L,), jnp.float32)
    shift16 = jnp.full((L,), SHIFT, jnp.int32)
    mask16 = jnp.full((L,), MASK, jnp.int32)
    off16 = jnp.full((L,), 1, jnp.int32) * core_off

    pltpu.sync_copy(coep, gbuf.at[pl.ds(0, 8)])
    vcoe = gbuf[0, pl.ds(0, L)]

    lanes = lax.iota(jnp.int32, L)

    def _csplat(i):
        sc = jnp.sum(jnp.where(lanes == i, vcoe, 0.0))
        return sc * jnp.full((L,), 1.0, jnp.float32)

    pltpu.sync_copy(packed.at[pl.ds(s * ROWS_E, ROWS_E)], pk_my)

    def _unpack(j, half, add_off):
        for g in range(ECH // L):
            sl = pl.ds(half * ECH + g * L, L)
            p = pk_my[j, sl]
            sv = lax.shift_right_logical(p, shift16)
            if add_off:
                sv = sv + off16
            sidx[pl.ds(g * L, L)] = sv
            didx[pl.ds(g * L, L)] = lax.bitwise_and(p, mask16)
        return 0

    def _fill_gbuf(val):
        def _row(r, _):
            for f in range(FV):
                gbuf[r, pl.ds(f * L, L)] = jnp.full((L,), val, jnp.float32)
            return 0
        lax.fori_loop(0, ECH, _row, 0)

    _fill_gbuf(0.0)

    def _zero_acc(k, _):
        pltpu.sync_copy(gbuf, acc.at[pl.ds(s * ROWS_PS + k * ECH, ECH)])
        return 0
    lax.fori_loop(0, ROWS_PS // ECH, _zero_acc, 0)
    plsc.subcore_barrier()

    _fill_gbuf(1.0)

    def _deg_chunk(j, _):
        for half in range(2):
            _unpack(j, half, False)
            pltpu.sync_copy(gbuf, acc.at[sidx], add=True)
        return 0
    lax.fori_loop(0, ROWS_E, _deg_chunk, 0)
    plsc.subcore_barrier()

    lane0 = lax.iota(jnp.int32, L) == 0

    def _dis_chunk(k, _):
        local = s * ROWS_PS + k * CCH
        pltpu.sync_copy(acc.at[pl.ds(local, CCH)], gbuf.at[pl.ds(0, CCH)])

        def _row(r, _2):
            d = gbuf[r, pl.ds(0, L)]
            iy = jnp.full((L,), 0x5F3759DF, jnp.int32) - (
                lax.shift_right_arithmetic(
                    plsc.bitcast(d, jnp.int32),
                    jnp.full((L,), 1, jnp.int32)))
            y = plsc.bitcast(iy, jnp.float32)
            for _i in range(3):
                y = y * (1.5 - 0.5 * d * y * y)
            y = jnp.where(d > 0.0, y, 0.0)
            plsc.store_scatter(dis, [jnp.full((L,), k * CCH + r, jnp.int32)],
                               y, mask=lane0)
            for f in range(FV):
                gbuf[r, pl.ds(f * L, L)] = zeros16
            return 0
        lax.fori_loop(0, CCH, _row, 0)
        pltpu.sync_copy(gbuf.at[pl.ds(0, CCH)], acc.at[pl.ds(local, CCH)])
        return 0
    lax.fori_loop(0, NCCH, _dis_chunk, 0)

    c0h = _csplat(0) * 0.5

    def _init_chunk(k, _):
        local = s * ROWS_PS + k * CCH
        g = core_off + local
        pltpu.sync_copy(h2.at[pl.ds(g, CCH)], gbuf.at[pl.ds(0, CCH)])

        def _row(r, _2):
            dsp = _splat(dis, k * CCH + r)
            for f in range(FV):
                sl = pl.ds(f * L, L)
                hvec = gbuf[r, sl]
                zv[r, sl] = dsp * hvec
                outv[r, sl] = c0h * hvec
            return 0
        lax.fori_loop(0, CCH, _row, 0)
        pltpu.sync_copy(zv, zA.at[pl.ds(g, CCH)])
        pltpu.sync_copy(outv, out2.at[pl.ds(g, CCH)])
        return 0
    lax.fori_loop(0, NCCH, _init_chunk, 0)
    plsc.subcore_barrier()

    for r in range(1, K + 1):
        zsrc = zA if (r % 2 == 1) else zB
        zdst = zB if (r % 2 == 1) else zA
        if r == 1:
            tx_rd, tx_wr = None, txQ
        elif r == 2:
            tx_rd, tx_wr = h2, txP
        else:
            tx_rd = tx_wr = (txQ if (r % 2 == 1) else txP)

        def _edge_chunk(j, _, zsrc=zsrc):
            for half in range(2):
                _unpack(j, half, True)
                pltpu.sync_copy(zsrc.at[sidx], gbuf)
                pltpu.sync_copy(gbuf, acc.at[didx], add=True)
            return 0
        lax.fori_loop(0, ROWS_E, _edge_chunk, 0)
        plsc.subcore_barrier()

        coer = _csplat(r)

        def _node_chunk(k, _, r=r, tx_rd=tx_rd, tx_wr=tx_wr, zdst=zdst,
                        coer=coer):
            local = s * ROWS_PS + k * CCH
            g = core_off + local
            pltpu.sync_copy(acc.at[pl.ds(local, CCH)], gbuf.at[pl.ds(0, CCH)])
            if r >= 2:
                pltpu.sync_copy(tx_rd.at[pl.ds(g, CCH)],
                                gbuf.at[pl.ds(CCH, CCH)])
            pltpu.sync_copy(out2.at[pl.ds(g, CCH)], outv)

            def _row(rr, _2):
                dsp = _splat(dis, k * CCH + rr)
                for f in range(FV):
                    sl = pl.ds(f * L, L)
                    a = gbuf[rr, sl]
                    if r == 1:
                        t2 = -(dsp * a)
                    else:
                        t2 = -2.0 * (dsp * a) - gbuf[CCH + rr, sl]
                    gbuf[CCH + rr, sl] = t2
                    outv[rr, sl] = outv[rr, sl] + coer * t2
                    if r < K:
                        zv[rr, sl] = dsp * t2
                    gbuf[rr, sl] = zeros16
                return 0
            lax.fori_loop(0, CCH, _row, 0)
            pltpu.sync_copy(gbuf.at[pl.ds(CCH, CCH)], tx_wr.at[pl.ds(g, CCH)])
            pltpu.sync_copy(outv, out2.at[pl.ds(g, CCH)])
            if r < K:
                pltpu.sync_copy(zv, zdst.at[pl.ds(g, CCH)])
                pltpu.sync_copy(gbuf.at[pl.ds(0, CCH)],
                                acc.at[pl.ds(local, CCH)])
            return 0
        lax.fori_loop(0, NCCH, _node_chunk, 0)
        if r < K:
            plsc.subcore_barrier()


def _sc_prop(h2, packed, coep):
    mesh = plsc.VectorSubcoreMesh(core_axis_name="c", subcore_axis_name="s")
    f32 = jnp.float32
    outs = pl.kernel(
        _sc_body,
        out_type=[jax.ShapeDtypeStruct((NC * NP, F), f32)] * 5,
        mesh=mesh,
        compiler_params=pltpu.CompilerParams(needs_layout_passes=False),
        scratch_types=[
            pltpu.VMEM_SHARED((NP, F), f32),
            pltpu.VMEM((ROWS_E, 2 * ECH), jnp.int32),
            pltpu.VMEM((ECH,), jnp.int32),
            pltpu.VMEM((ECH,), jnp.int32),
            pltpu.VMEM((ECH, F), f32),
            pltpu.VMEM((CCH, F), f32),
            pltpu.VMEM((CCH, F), f32),
            pltpu.VMEM((ROWS_PS,), f32),
        ],
    )(h2, packed, coep)
    return outs[0]


def kernel(x, edge_index, W, b, temp):
    src = edge_index[0].astype(jnp.int32)
    dst = edge_index[1].astype(jnp.int32)
    pad = jnp.full((EP - E,), PAD_IDX, jnp.int32)
    srcp = jnp.concatenate([src, pad])
    dstp = jnp.concatenate([dst, pad])
    packed = (srcp * (MASK + 1) + dstp).reshape(ROWS_E * NS, 2 * ECH)
    x_pad = jnp.concatenate(
        [x, jnp.zeros((NP - N, D), jnp.float32)], axis=0)
    b2 = b.reshape(1, D)
    temp8 = jnp.zeros((8, 128), jnp.float32).at[0, : K + 1].set(temp)

    h2 = _matmul_h2(x_pad, W, b2)
    coep = _coe_tc(temp8, jnp.asarray(_CHEB_T))
    out2 = _sc_prop(h2, packed, coep)
    out = _epilogue(out2)
    return out[:N]


# speedup vs baseline: 2.9506x; 1.1393x over previous
import math

import jax
import jax.numpy as jnp
import numpy as np
from jax import lax
from jax.experimental import pallas as pl
from jax.experimental.pallas import tpu as pltpu
from jax.experimental.pallas import tpu_sc as plsc

K = 10
N = 10000
E = 160000
D = 256

NC = 2
NS = 16
L = 16
F = D // NC
FV = F // L

NP = 10240
ROWS_PS = NP // NS
CCH = 32
NCCH = ROWS_PS // CCH

ECH = 64
ROWS_E = 80
EP = ROWS_E * NS * 2 * ECH
PAD_IDX = N
SHIFT = 14
MASK = (1 << SHIFT) - 1


def _cheby(i, x):
    if i == 0:
        return 1.0
    if i == 1:
        return x
    t0, t1 = 1.0, x
    for _ in range(2, i + 1):
        t0, t1 = t1, 2.0 * x * t1 - t0
    return t1


def _cheb_t_padded():
    xs = [math.cos((K - j + 0.5) * math.pi / (K + 1)) for j in range(K + 1)]
    Ct = np.zeros((128, 128), dtype=np.float32)
    for i in range(K + 1):
        for j in range(K + 1):
            Ct[j, i] = _cheby(i, xs[j])
    return Ct


_CHEB_T = _cheb_t_padded()


def _mm_body(x_ref, w_ref, b_ref, o_ref):
    o_ref[...] = (
        lax.dot_general(
            x_ref[...], w_ref[...], (((1,), (1,)), ((), ())),
            preferred_element_type=jnp.float32,
        )
        + b_ref[...]
    )


def _matmul_h2(x_pad, W, b2):
    nb = NP // 128
    return pl.pallas_call(
        _mm_body,
        grid=(nb, NC),
        in_specs=[
            pl.BlockSpec((128, D), lambda i, c: (i, 0)),
            pl.BlockSpec((F, D), lambda i, c: (c, 0)),
            pl.BlockSpec((1, F), lambda i, c: (0, c)),
        ],
        out_specs=pl.BlockSpec((128, F), lambda i, c: (c * nb + i, 0)),
        out_shape=jax.ShapeDtypeStruct((NC * NP, F), jnp.float32),
    )(x_pad, W, b2)


def _coe_body(t_ref, c_ref, o_ref):
    tr = jnp.maximum(t_ref[...], 0.0)
    o_ref[...] = lax.dot_general(
        tr, c_ref[...], (((1,), (0,)), ((), ())),
        preferred_element_type=jnp.float32,
    ) * (2.0 / (K + 1))


def _coe_tc(temp8, chebT):
    return pl.pallas_call(
        _coe_body,
        out_shape=jax.ShapeDtypeStruct((8, 128), jnp.float32),
    )(temp8, chebT)


def _ep_body(a_ref, b_ref, o_ref):
    o_ref[:, :F] = jnp.maximum(a_ref[...], 0.0)
    o_ref[:, F:] = jnp.maximum(b_ref[...], 0.0)


def _epilogue(out2):
    nb = NP // 128
    return pl.pallas_call(
        _ep_body,
        grid=(nb,),
        in_specs=[
            pl.BlockSpec((128, F), lambda i: (i, 0)),
            pl.BlockSpec((128, F), lambda i: (nb + i, 0)),
        ],
        out_specs=pl.BlockSpec((128, D), lambda i: (i, 0)),
        out_shape=jax.ShapeDtypeStruct((NP, D), jnp.float32),
    )(out2, out2)


def _splat(ref, idx):
    return plsc.load_gather(ref, [jnp.full((L,), idx, jnp.int32)])


def _sc_body(h2, packed, coep,
             out2, txP, txQ, zA, zB,
             acc,
             pk_my, sidx, didx, sidxB, didxB, gbuf, gbufB, outv, zv,
             dis, semA, semB, semR, semW):
    c = lax.axis_index("c")
    s = lax.axis_index("s")
    core_off = c * NP
    zeros16 = jnp.zeros((L,), jnp.float32)
    shift16 = jnp.full((L,), SHIFT, jnp.int32)
    mask16 = jnp.full((L,), MASK, jnp.int32)
    off16 = jnp.full((L,), 1, jnp.int32) * core_off

    pltpu.sync_copy(coep, gbuf.at[pl.ds(0, 8)])
    vcoe = gbuf[0, pl.ds(0, L)]

    lanes = lax.iota(jnp.int32, L)

    def _csplat(i):
        sc = jnp.sum(jnp.where(lanes == i, vcoe, 0.0))
        return sc * jnp.full((L,), 1.0, jnp.float32)

    pltpu.sync_copy(packed.at[pl.ds(s * ROWS_E, ROWS_E)], pk_my)

    def _unpack(j, half, add_off, sb, db):
        for g in range(ECH // L):
            sl = pl.ds(half * ECH + g * L, L)
            p = pk_my[j, sl]
            sv = lax.shift_right_logical(p, shift16)
            if add_off:
                sv = sv + off16
            sb[pl.ds(g * L, L)] = sv
            db[pl.ds(g * L, L)] = lax.bitwise_and(p, mask16)
        return 0

    def _fill_gbuf(val):
        def _row(r, _):
            for f in range(FV):
                gbuf[r, pl.ds(f * L, L)] = jnp.full((L,), val, jnp.float32)
            return 0
        lax.fori_loop(0, ECH, _row, 0)

    _fill_gbuf(0.0)

    def _zero_acc(k, _):
        pltpu.sync_copy(gbuf, acc.at[pl.ds(s * ROWS_PS + k * ECH, ECH)])
        return 0
    lax.fori_loop(0, ROWS_PS // ECH, _zero_acc, 0)
    plsc.subcore_barrier()

    _fill_gbuf(1.0)

    def _deg_chunk(j, _):
        for half in range(2):
            _unpack(j, half, False, sidx, didx)
            pltpu.sync_copy(gbuf, acc.at[sidx], add=True)
        return 0
    lax.fori_loop(0, ROWS_E, _deg_chunk, 0)
    plsc.subcore_barrier()

    lane0 = lax.iota(jnp.int32, L) == 0

    def _dis_chunk(k, _):
        local = s * ROWS_PS + k * CCH
        pltpu.sync_copy(acc.at[pl.ds(local, CCH)], gbuf.at[pl.ds(0, CCH)])

        def _row(r, _2):
            d = gbuf[r, pl.ds(0, L)]
            iy = jnp.full((L,), 0x5F3759DF, jnp.int32) - (
                lax.shift_right_arithmetic(
                    plsc.bitcast(d, jnp.int32),
                    jnp.full((L,), 1, jnp.int32)))
            y = plsc.bitcast(iy, jnp.float32)
            for _i in range(3):
                y = y * (1.5 - 0.5 * d * y * y)
            y = jnp.where(d > 0.0, y, 0.0)
            plsc.store_scatter(dis, [jnp.full((L,), k * CCH + r, jnp.int32)],
                               y, mask=lane0)
            for f in range(FV):
                gbuf[r, pl.ds(f * L, L)] = zeros16
            return 0
        lax.fori_loop(0, CCH, _row, 0)
        pltpu.sync_copy(gbuf.at[pl.ds(0, CCH)], acc.at[pl.ds(local, CCH)])
        return 0
    lax.fori_loop(0, NCCH, _dis_chunk, 0)

    c0h = _csplat(0) * 0.5

    def _init_chunk(k, _):
        local = s * ROWS_PS + k * CCH
        g = core_off + local
        pltpu.sync_copy(h2.at[pl.ds(g, CCH)], gbuf.at[pl.ds(0, CCH)])

        def _row(r, _2):
            dsp = _splat(dis, k * CCH + r)
            for f in range(FV):
                sl = pl.ds(f * L, L)
                hvec = gbuf[r, sl]
                zv[r, sl] = dsp * hvec
                outv[r, sl] = c0h * hvec
            return 0
        lax.fori_loop(0, CCH, _row, 0)
        pltpu.sync_copy(zv, zA.at[pl.ds(g, CCH)])
        pltpu.sync_copy(outv, out2.at[pl.ds(g, CCH)])
        return 0
    lax.fori_loop(0, NCCH, _init_chunk, 0)
    plsc.subcore_barrier()

    for r in range(1, K + 1):
        zsrc = zA if (r % 2 == 1) else zB
        zdst = zB if (r % 2 == 1) else zA
        if r == 1:
            tx_rd, tx_wr = None, txQ
        elif r == 2:
            tx_rd, tx_wr = h2, txP
        else:
            tx_rd = tx_wr = (txQ if (r % 2 == 1) else txP)

        def _edge_chunk(j, _, zsrc=zsrc):
            _unpack(j, 0, True, sidx, didx)
            dA = pltpu.async_copy(zsrc.at[sidx], gbuf, semA)
            _unpack(j, 1, True, sidxB, didxB)
            dB = pltpu.async_copy(zsrc.at[sidxB], gbufB, semA)
            dA.wait()
            pltpu.sync_copy(gbuf, acc.at[didx], add=True)
            dB.wait()
            pltpu.sync_copy(gbufB, acc.at[didxB], add=True)
            return 0
        lax.fori_loop(0, ROWS_E, _edge_chunk, 0)
        plsc.subcore_barrier()

        coer = _csplat(r)

        def _node_chunk(k, _, r=r, tx_rd=tx_rd, tx_wr=tx_wr, zdst=zdst,
                        coer=coer):
            local = s * ROWS_PS + k * CCH
            g = core_off + local
            pltpu.sync_copy(acc.at[pl.ds(local, CCH)], gbuf.at[pl.ds(0, CCH)])
            if r >= 2:
                pltpu.sync_copy(tx_rd.at[pl.ds(g, CCH)],
                                gbuf.at[pl.ds(CCH, CCH)])
            pltpu.sync_copy(out2.at[pl.ds(g, CCH)], outv)

            def _row(rr, _2):
                dsp = _splat(dis, k * CCH + rr)
                for f in range(FV):
                    sl = pl.ds(f * L, L)
                    a = gbuf[rr, sl]
                    if r == 1:
                        t2 = -(dsp * a)
                    else:
                        t2 = -2.0 * (dsp * a) - gbuf[CCH + rr, sl]
                    gbufB[rr, sl] = t2
                    outv[rr, sl] = outv[rr, sl] + coer * t2
                    if r < K:
                        zv[rr, sl] = dsp * t2
                    gbuf[rr, sl] = zeros16
                return 0
            lax.fori_loop(0, CCH, _row, 0)
            pltpu.sync_copy(gbufB.at[pl.ds(0, CCH)], tx_wr.at[pl.ds(g, CCH)])
            pltpu.sync_copy(outv, out2.at[pl.ds(g, CCH)])
            if r < K:
                pltpu.sync_copy(zv, zdst.at[pl.ds(g, CCH)])
                pltpu.sync_copy(gbuf.at[pl.ds(0, CCH)],
                                acc.at[pl.ds(local, CCH)])
            return 0
        lax.fori_loop(0, NCCH, _node_chunk, 0)
        if r < K:
            plsc.subcore_barrier()


def _sc_prop(h2, packed, coep):
    mesh = plsc.VectorSubcoreMesh(core_axis_name="c", subcore_axis_name="s")
    f32 = jnp.float32
    outs = pl.kernel(
        _sc_body,
        out_type=[jax.ShapeDtypeStruct((NC * NP, F), f32)] * 5,
        mesh=mesh,
        compiler_params=pltpu.CompilerParams(needs_layout_passes=False),
        scratch_types=[
            pltpu.VMEM_SHARED((NP, F), f32),
            pltpu.VMEM((ROWS_E, 2 * ECH), jnp.int32),
            pltpu.VMEM((ECH,), jnp.int32),
            pltpu.VMEM((ECH,), jnp.int32),
            pltpu.VMEM((ECH,), jnp.int32),
            pltpu.VMEM((ECH,), jnp.int32),
            pltpu.VMEM((ECH, F), f32),
            pltpu.VMEM((ECH, F), f32),
            pltpu.VMEM((CCH, F), f32),
            pltpu.VMEM((CCH, F), f32),
            pltpu.VMEM((ROWS_PS,), f32),
            pltpu.SemaphoreType.DMA,
            pltpu.SemaphoreType.DMA,
            pltpu.SemaphoreType.DMA,
            pltpu.SemaphoreType.DMA,
        ],
    )(h2, packed, coep)
    return outs[0]


def kernel(x, edge_index, W, b, temp):
    src = edge_index[0].astype(jnp.int32)
    dst = edge_index[1].astype(jnp.int32)
    pad = jnp.full((EP - E,), PAD_IDX, jnp.int32)
    srcp = jnp.concatenate([src, pad])
    dstp = jnp.concatenate([dst, pad])
    packed = (srcp * (MASK + 1) + dstp).reshape(ROWS_E * NS, 2 * ECH)
    x_pad = jnp.concatenate(
        [x, jnp.zeros((NP - N, D), jnp.float32)], axis=0)
    b2 = b.reshape(1, D)
    temp8 = jnp.zeros((8, 128), jnp.float32).at[0, : K + 1].set(temp)

    h2 = _matmul_h2(x_pad, W, b2)
    coep = _coe_tc(temp8, jnp.asarray(_CHEB_T))
    out2 = _sc_prop(h2, packed, coep)
    out = _epilogue(out2)
    return out[:N]


# speedup vs baseline: 2.9593x; 1.0029x over previous
import math

import jax
import jax.numpy as jnp
import numpy as np
from jax import lax
from jax.experimental import pallas as pl
from jax.experimental.pallas import tpu as pltpu
from jax.experimental.pallas import tpu_sc as plsc

K = 10
N = 10000
E = 160000
D = 256

NC = 2
NS = 16
L = 16
F = D // NC
FV = F // L

NP = 10240
ROWS_PS = NP // NS
CCH = 32
NCCH = ROWS_PS // CCH

ECH = 64
ROWS_E = 80
EP = ROWS_E * NS * 2 * ECH
PAD_IDX = N
SHIFT = 14
MASK = (1 << SHIFT) - 1


def _cheby(i, x):
    if i == 0:
        return 1.0
    if i == 1:
        return x
    t0, t1 = 1.0, x
    for _ in range(2, i + 1):
        t0, t1 = t1, 2.0 * x * t1 - t0
    return t1


def _cheb_t_padded():
    xs = [math.cos((K - j + 0.5) * math.pi / (K + 1)) for j in range(K + 1)]
    Ct = np.zeros((128, 128), dtype=np.float32)
    for i in range(K + 1):
        for j in range(K + 1):
            Ct[j, i] = _cheby(i, xs[j])
    return Ct


_CHEB_T = _cheb_t_padded()


def _mm_body(x_ref, w_ref, b_ref, o_ref):
    o_ref[...] = (
        lax.dot_general(
            x_ref[...], w_ref[...], (((1,), (1,)), ((), ())),
            preferred_element_type=jnp.float32,
        )
        + b_ref[...]
    )


def _matmul_h2(x_pad, W, b2):
    nb = NP // 128
    return pl.pallas_call(
        _mm_body,
        grid=(nb, NC),
        in_specs=[
            pl.BlockSpec((128, D), lambda i, c: (i, 0)),
            pl.BlockSpec((F, D), lambda i, c: (c, 0)),
            pl.BlockSpec((1, F), lambda i, c: (0, c)),
        ],
        out_specs=pl.BlockSpec((128, F), lambda i, c: (c * nb + i, 0)),
        out_shape=jax.ShapeDtypeStruct((NC * NP, F), jnp.float32),
    )(x_pad, W, b2)


def _coe_body(t_ref, c_ref, o_ref):
    tr = jnp.maximum(t_ref[...], 0.0)
    o_ref[...] = lax.dot_general(
        tr, c_ref[...], (((1,), (0,)), ((), ())),
        preferred_element_type=jnp.float32,
    ) * (2.0 / (K + 1))


def _coe_tc(temp8, chebT):
    return pl.pallas_call(
        _coe_body,
        out_shape=jax.ShapeDtypeStruct((8, 128), jnp.float32),
    )(temp8, chebT)


def _ep_body(a_ref, b_ref, o_ref):
    o_ref[:, :F] = jnp.maximum(a_ref[...], 0.0)
    o_ref[:, F:] = jnp.maximum(b_ref[...], 0.0)


def _epilogue(out2):
    nb = NP // 128
    return pl.pallas_call(
        _ep_body,
        grid=(nb,),
        in_specs=[
            pl.BlockSpec((128, F), lambda i: (i, 0)),
            pl.BlockSpec((128, F), lambda i: (nb + i, 0)),
        ],
        out_specs=pl.BlockSpec((128, D), lambda i: (i, 0)),
        out_shape=jax.ShapeDtypeStruct((NP, D), jnp.float32),
    )(out2, out2)


def _splat(ref, idx):
    return plsc.load_gather(ref, [jnp.full((L,), idx, jnp.int32)])


def _sc_body(h2, packed, coep,
             out2, txP, txQ, zA, zB,
             acc,
             pk_my, sidx, didx, sidxB, didxB, gbuf, gbufB, outv, zv,
             dis, semA, semB, semR, semW):
    c = lax.axis_index("c")
    s = lax.axis_index("s")
    core_off = c * NP
    zeros16 = jnp.zeros((L,), jnp.float32)
    shift16 = jnp.full((L,), SHIFT, jnp.int32)
    mask16 = jnp.full((L,), MASK, jnp.int32)
    off16 = jnp.full((L,), 1, jnp.int32) * core_off

    pltpu.sync_copy(coep, gbuf.at[pl.ds(0, 8)])
    vcoe = gbuf[0, pl.ds(0, L)]

    lanes = lax.iota(jnp.int32, L)

    def _csplat(i):
        sc = jnp.sum(jnp.where(lanes == i, vcoe, 0.0))
        return sc * jnp.full((L,), 1.0, jnp.float32)

    pltpu.sync_copy(packed.at[pl.ds(s * ROWS_E, ROWS_E)], pk_my)

    def _unpack(j, half, add_off, sb, db):
        for g in range(ECH // L):
            sl = pl.ds(half * ECH + g * L, L)
            p = pk_my[j, sl]
            sv = lax.shift_right_logical(p, shift16)
            if add_off:
                sv = sv + off16
            sb[pl.ds(g * L, L)] = sv
            db[pl.ds(g * L, L)] = lax.bitwise_and(p, mask16)
        return 0

    def _fill_gbuf(val):
        def _row(r, _):
            for f in range(FV):
                gbuf[r, pl.ds(f * L, L)] = jnp.full((L,), val, jnp.float32)
            return 0
        lax.fori_loop(0, ECH, _row, 0)

    _fill_gbuf(0.0)

    def _zero_acc(k, _):
        pltpu.sync_copy(gbuf, acc.at[pl.ds(s * ROWS_PS + k * ECH, ECH)])
        return 0
    lax.fori_loop(0, ROWS_PS // ECH, _zero_acc, 0)
    plsc.subcore_barrier()

    _fill_gbuf(1.0)

    def _deg_chunk(j, _):
        for half in range(2):
            _unpack(j, half, False, sidx, didx)
            pltpu.sync_copy(gbuf, acc.at[sidx], add=True)
        return 0
    lax.fori_loop(0, ROWS_E, _deg_chunk, 0)
    plsc.subcore_barrier()

    lane0 = lax.iota(jnp.int32, L) == 0

    def _dis_chunk(k, _):
        local = s * ROWS_PS + k * CCH
        pltpu.sync_copy(acc.at[pl.ds(local, CCH)], gbuf.at[pl.ds(0, CCH)])

        def _row(r, _2):
            d = gbuf[r, pl.ds(0, L)]
            iy = jnp.full((L,), 0x5F3759DF, jnp.int32) - (
                lax.shift_right_arithmetic(
                    plsc.bitcast(d, jnp.int32),
                    jnp.full((L,), 1, jnp.int32)))
            y = plsc.bitcast(iy, jnp.float32)
            for _i in range(3):
                y = y * (1.5 - 0.5 * d * y * y)
            y = jnp.where(d > 0.0, y, 0.0)
            plsc.store_scatter(dis, [jnp.full((L,), k * CCH + r, jnp.int32)],
                               y, mask=lane0)
            for f in range(FV):
                gbuf[r, pl.ds(f * L, L)] = zeros16
            return 0
        lax.fori_loop(0, CCH, _row, 0)
        pltpu.sync_copy(gbuf.at[pl.ds(0, CCH)], acc.at[pl.ds(local, CCH)])
        return 0
    lax.fori_loop(0, NCCH, _dis_chunk, 0)

    c0h = _csplat(0) * 0.5

    def _init_chunk(k, _):
        local = s * ROWS_PS + k * CCH
        g = core_off + local
        pltpu.sync_copy(h2.at[pl.ds(g, CCH)], gbuf.at[pl.ds(0, CCH)])

        def _row(r, _2):
            dsp = _splat(dis, k * CCH + r)
            for f in range(FV):
                sl = pl.ds(f * L, L)
                hvec = gbuf[r, sl]
                zv[r, sl] = dsp * hvec
                outv[r, sl] = c0h * hvec
            return 0
        lax.fori_loop(0, CCH, _row, 0)
        pltpu.sync_copy(zv, zA.at[pl.ds(g, CCH)])
        pltpu.sync_copy(outv, out2.at[pl.ds(g, CCH)])
        return 0
    lax.fori_loop(0, NCCH, _init_chunk, 0)
    plsc.subcore_barrier()

    for r in range(1, K + 1):
        zsrc = zA if (r % 2 == 1) else zB
        zdst = zB if (r % 2 == 1) else zA
        if r == 1:
            tx_rd, tx_wr = None, txQ
        elif r == 2:
            tx_rd, tx_wr = h2, txP
        else:
            tx_rd = tx_wr = (txQ if (r % 2 == 1) else txP)

        def _edge_chunk(j, _, zsrc=zsrc):
            _unpack(j, 0, True, sidx, didx)
            dA = pltpu.async_copy(zsrc.at[sidx], gbuf, semA)
            _unpack(j, 1, True, sidxB, didxB)
            dB = pltpu.async_copy(zsrc.at[sidxB], gbufB, semA)
            dA.wait()
            sA = pltpu.async_copy(gbuf, acc.at[didx], semB, add=True)
            dB.wait()
            pltpu.sync_copy(gbufB, acc.at[didxB], add=True)
            sA.wait()
            return 0
        lax.fori_loop(0, ROWS_E, _edge_chunk, 0)
        plsc.subcore_barrier()

        coer = _csplat(r)

        def _node_chunk(k, _, r=r, tx_rd=tx_rd, tx_wr=tx_wr, zdst=zdst,
                        coer=coer):
            local = s * ROWS_PS + k * CCH
            g = core_off + local
            pltpu.sync_copy(acc.at[pl.ds(local, CCH)], gbuf.at[pl.ds(0, CCH)])
            if r >= 2:
                pltpu.sync_copy(tx_rd.at[pl.ds(g, CCH)],
                                gbuf.at[pl.ds(CCH, CCH)])
            pltpu.sync_copy(out2.at[pl.ds(g, CCH)], outv)

            def _row(rr, _2):
                dsp = _splat(dis, k * CCH + rr)
                for f in range(FV):
                    sl = pl.ds(f * L, L)
                    a = gbuf[rr, sl]
                    if r == 1:
                        t2 = -(dsp * a)
                    else:
                        t2 = -2.0 * (dsp * a) - gbuf[CCH + rr, sl]
                    gbufB[rr, sl] = t2
                    outv[rr, sl] = outv[rr, sl] + coer * t2
                    if r < K:
                        zv[rr, sl] = dsp * t2
                    gbuf[rr, sl] = zeros16
                return 0
            lax.fori_loop(0, CCH, _row, 0)
            pltpu.sync_copy(gbufB.at[pl.ds(0, CCH)], tx_wr.at[pl.ds(g, CCH)])
            pltpu.sync_copy(outv, out2.at[pl.ds(g, CCH)])
            if r < K:
                pltpu.sync_copy(zv, zdst.at[pl.ds(g, CCH)])
                pltpu.sync_copy(gbuf.at[pl.ds(0, CCH)],
                                acc.at[pl.ds(local, CCH)])
            return 0
        lax.fori_loop(0, NCCH, _node_chunk, 0)
        if r < K:
            plsc.subcore_barrier()


def _sc_prop(h2, packed, coep):
    mesh = plsc.VectorSubcoreMesh(core_axis_name="c", subcore_axis_name="s")
    f32 = jnp.float32
    outs = pl.kernel(
        _sc_body,
        out_type=[jax.ShapeDtypeStruct((NC * NP, F), f32)] * 5,
        mesh=mesh,
        compiler_params=pltpu.CompilerParams(needs_layout_passes=False),
        scratch_types=[
            pltpu.VMEM_SHARED((NP, F), f32),
            pltpu.VMEM((ROWS_E, 2 * ECH), jnp.int32),
            pltpu.VMEM((ECH,), jnp.int32),
            pltpu.VMEM((ECH,), jnp.int32),
            pltpu.VMEM((ECH,), jnp.int32),
            pltpu.VMEM((ECH,), jnp.int32),
            pltpu.VMEM((ECH, F), f32),
            pltpu.VMEM((ECH, F), f32),
            pltpu.VMEM((CCH, F), f32),
            pltpu.VMEM((CCH, F), f32),
            pltpu.VMEM((ROWS_PS,), f32),
            pltpu.SemaphoreType.DMA,
            pltpu.SemaphoreType.DMA,
            pltpu.SemaphoreType.DMA,
            pltpu.SemaphoreType.DMA,
        ],
    )(h2, packed, coep)
    return outs[0]


def kernel(x, edge_index, W, b, temp):
    src = edge_index[0].astype(jnp.int32)
    dst = edge_index[1].astype(jnp.int32)
    pad = jnp.full((EP - E,), PAD_IDX, jnp.int32)
    srcp = jnp.concatenate([src, pad])
    dstp = jnp.concatenate([dst, pad])
    packed = (srcp * (MASK + 1) + dstp).reshape(ROWS_E * NS, 2 * ECH)
    x_pad = jnp.concatenate(
        [x, jnp.zeros((NP - N, D), jnp.float32)], axis=0)
    b2 = b.reshape(1, D)
    temp8 = jnp.zeros((8, 128), jnp.float32).at[0, : K + 1].set(temp)

    h2 = _matmul_h2(x_pad, W, b2)
    coep = _coe_tc(temp8, jnp.asarray(_CHEB_T))
    out2 = _sc_prop(h2, packed, coep)
    out = _epilogue(out2)
    return out[:N]
